# Initial kernel scaffold; baseline (speedup 1.0000x reference)
import jax
import jax.numpy as jnp
from jax.experimental import pallas as pl


def kernel(x, edge_index, W1, b1, Wmu, bmu, Wls, bls, eps):
    raise NotImplementedError("write your pallas kernel here")


# speedup vs baseline: 11.9794x; 11.9794x over previous
import functools

import jax
import jax.numpy as jnp
from jax import lax
from jax.experimental import pallas as pl
from jax.experimental.pallas import tpu as pltpu
from jax.experimental.pallas import tpu_sc as plsc

N = 10000
E = 160000
D_IN = 128
D_H = 64
D_Z = 16

NW = 32
BATCH = 128
NBATCH = 40
EPAD = NW * NBATCH * BATCH
NPAD = 10112
TROWS = NPAD // 16

MB = 1000
CB = 2048


def _make_deg_kernel():
    mesh = plsc.VectorSubcoreMesh(core_axis_name="c", subcore_axis_name="s")

    @functools.partial(
        pl.kernel,
        out_type=jax.ShapeDtypeStruct((2, NPAD, 16), jnp.float32),
        mesh=mesh,
        scratch_types=[
            pltpu.VMEM((NBATCH, BATCH), jnp.int32),
            pltpu.VMEM((BATCH, 16), jnp.float32),
            pltpu.VMEM_SHARED((NPAD, 16), jnp.float32),
        ],
        compiler_params=pltpu.CompilerParams(use_tc_tiling_on_sc=False),
    )
    def deg_kernel(dst_hbm, ones_hbm, zero_hbm, out_hbm, didx, ones_v, acc):
        c = lax.axis_index("c")
        s = lax.axis_index("s")
        w = c * 16 + s
        r0 = s * TROWS
        pltpu.sync_copy(zero_hbm, acc.at[pl.ds(r0, TROWS)])
        pltpu.sync_copy(dst_hbm.at[pl.ds(w * NBATCH, NBATCH)], didx)
        pltpu.sync_copy(ones_hbm, ones_v)
        plsc.subcore_barrier()

        def body(j, carry):
            pltpu.sync_copy(ones_v, acc.at[didx.at[j]], add=True)
            return carry

        lax.fori_loop(0, NBATCH, body, 0)
        plsc.subcore_barrier()
        pltpu.sync_copy(acc.at[pl.ds(r0, TROWS)],
                        out_hbm.at[c, pl.ds(r0, TROWS)])

    return deg_kernel


def _make_agg_kernel(d):
    mesh = plsc.VectorSubcoreMesh(core_axis_name="c", subcore_axis_name="s")

    @functools.partial(
        pl.kernel,
        out_type=jax.ShapeDtypeStruct((2, NPAD, d), jnp.float32),
        mesh=mesh,
        scratch_types=[
            pltpu.VMEM((NBATCH, BATCH), jnp.int32),
            pltpu.VMEM((NBATCH, BATCH), jnp.int32),
            pltpu.VMEM((BATCH, d), jnp.float32),
            pltpu.VMEM_SHARED((NPAD, d), jnp.float32),
            pltpu.SemaphoreType.DMA,
        ],
        compiler_params=pltpu.CompilerParams(use_tc_tiling_on_sc=False),
    )
    def agg_kernel(hs_hbm, src_hbm, dst_hbm, zero_hbm, out_hbm,
                   sidx, didx, rows, acc, sem):
        c = lax.axis_index("c")
        s = lax.axis_index("s")
        w = c * 16 + s
        r0 = s * TROWS
        pltpu.sync_copy(zero_hbm, acc.at[pl.ds(r0, TROWS)])
        pltpu.sync_copy(src_hbm.at[pl.ds(w * NBATCH, NBATCH)], sidx)
        pltpu.sync_copy(dst_hbm.at[pl.ds(w * NBATCH, NBATCH)], didx)
        plsc.subcore_barrier()

        def body(j, carry):
            pltpu.async_copy(hs_hbm.at[sidx.at[j]], rows, sem).wait()
            pltpu.sync_copy(rows, acc.at[didx.at[j]], add=True)
            return carry

        lax.fori_loop(0, NBATCH, body, 0)
        plsc.subcore_barrier()
        pltpu.sync_copy(acc.at[pl.ds(r0, TROWS)],
                        out_hbm.at[c, pl.ds(r0, TROWS)])

    return agg_kernel


_deg_call = _make_deg_kernel()
_agg64_call = _make_agg_kernel(D_H)
_agg32_call = _make_agg_kernel(2 * D_Z)


def _tc1_body(x_ref, w_ref, b_ref, p0_ref, p1_ref, hs_ref, dinv_ref):
    deg = p0_ref[:, 0:1] + p1_ref[:, 0:1] + 1.0
    dinv = lax.rsqrt(deg)
    h = jnp.dot(x_ref[...], w_ref[...], preferred_element_type=jnp.float32)
    hs_ref[...] = (h + b_ref[...]) * dinv
    dinv_ref[...] = jnp.broadcast_to(dinv, (dinv.shape[0], 16))


def _tc2_body(a0_ref, a1_ref, hs_ref, dinv_ref, w_ref, b_ref, gs_ref):
    dinv = dinv_ref[:, 0:1]
    h1 = jnp.maximum((a0_ref[...] + a1_ref[...] + hs_ref[...]) * dinv, 0.0)
    g = jnp.dot(h1, w_ref[...], preferred_element_type=jnp.float32)
    gs_ref[...] = (g + b_ref[...]) * dinv


def _tc3_body(q0_ref, q1_ref, gs_ref, dinv_ref, eps_ref,
              mu_ref, ls_ref, z_ref):
    m = (q0_ref[...] + q1_ref[...] + gs_ref[...]) * dinv_ref[:, 0:1]
    mu = m[:, :D_Z]
    ls = m[:, D_Z:]
    mu_ref[...] = mu
    ls_ref[...] = ls
    z_ref[...] = mu + eps_ref[...] * jnp.exp(ls)


def _dec_body(zr_ref, zc_ref, out_ref):
    out_ref[...] = lax.dot_general(
        zr_ref[...], zc_ref[...], (((1,), (1,)), ((), ())),
        preferred_element_type=jnp.float32)


def _node_spec(d):
    return pl.BlockSpec((MB, d), lambda i: (i, 0))


def _full_spec(shape):
    return pl.BlockSpec(shape, lambda i: (0,) * len(shape))


def kernel(x, edge_index, W1, b1, Wmu, bmu, Wls, bls, eps):
    src = edge_index[0].astype(jnp.int32)
    dst = edge_index[1].astype(jnp.int32)
    pad = jnp.full((EPAD - E,), N, dtype=jnp.int32)
    srcp = jnp.concatenate([src, pad]).reshape(EPAD // BATCH, BATCH)
    dstp = jnp.concatenate([dst, pad]).reshape(EPAD // BATCH, BATCH)

    ones16 = jnp.ones((BATCH, 16), jnp.float32)
    zero16 = jnp.zeros((TROWS, 16), jnp.float32)
    zero64 = jnp.zeros((TROWS, D_H), jnp.float32)
    zero32 = jnp.zeros((TROWS, 2 * D_Z), jnp.float32)

    degp = _deg_call(dstp, ones16, zero16)
    p0 = degp[0, :N]
    p1 = degp[1, :N]

    grid = N // MB
    hs, dinv = pl.pallas_call(
        _tc1_body,
        grid=(grid,),
        in_specs=[_node_spec(D_IN), _full_spec((D_IN, D_H)),
                  _full_spec((1, D_H)), _node_spec(16), _node_spec(16)],
        out_specs=[_node_spec(D_H), _node_spec(16)],
        out_shape=[jax.ShapeDtypeStruct((N, D_H), jnp.float32),
                   jax.ShapeDtypeStruct((N, 16), jnp.float32)],
    )(x, W1, b1.reshape(1, D_H), p0, p1)

    hs_pad = jnp.pad(hs, ((0, NPAD - N), (0, 0)))
    agg1 = _agg64_call(hs_pad, srcp, dstp, zero64)
    a0 = agg1[0, :N]
    a1 = agg1[1, :N]

    Wc = jnp.concatenate([Wmu, Wls], axis=1)
    bc = jnp.concatenate([bmu, bls]).reshape(1, 2 * D_Z)
    gs = pl.pallas_call(
        _tc2_body,
        grid=(grid,),
        in_specs=[_node_spec(D_H), _node_spec(D_H), _node_spec(D_H),
                  _node_spec(16), _full_spec((D_H, 2 * D_Z)),
                  _full_spec((1, 2 * D_Z))],
        out_specs=_node_spec(2 * D_Z),
        out_shape=jax.ShapeDtypeStruct((N, 2 * D_Z), jnp.float32),
    )(a0, a1, hs, dinv, Wc, bc)

    gs_pad = jnp.pad(gs, ((0, NPAD - N), (0, 0)))
    agg2 = _agg32_call(gs_pad, srcp, dstp, zero32)
    q0 = agg2[0, :N]
    q1 = agg2[1, :N]

    mu, logstd, z = pl.pallas_call(
        _tc3_body,
        grid=(grid,),
        in_specs=[_node_spec(2 * D_Z), _node_spec(2 * D_Z),
                  _node_spec(2 * D_Z), _node_spec(16), _node_spec(D_Z)],
        out_specs=[_node_spec(D_Z), _node_spec(D_Z), _node_spec(D_Z)],
        out_shape=[jax.ShapeDtypeStruct((N, D_Z), jnp.float32),
                   jax.ShapeDtypeStruct((N, D_Z), jnp.float32),
                   jax.ShapeDtypeStruct((N, D_Z), jnp.float32)],
    )(q0, q1, gs, dinv, eps)

    adj = pl.pallas_call(
        _dec_body,
        grid=(N // MB, pl.cdiv(N, CB)),
        in_specs=[pl.BlockSpec((MB, D_Z), lambda i, j: (i, 0)),
                  pl.BlockSpec((CB, D_Z), lambda i, j: (j, 0))],
        out_specs=pl.BlockSpec((MB, CB), lambda i, j: (i, j)),
        out_shape=jax.ShapeDtypeStruct((N, N), jnp.float32),
    )(z, z)

    return (adj, mu, logstd)


# speedup vs baseline: 12.4114x; 1.0361x over previous
import functools

import jax
import jax.numpy as jnp
from jax import lax
from jax.experimental import pallas as pl
from jax.experimental.pallas import tpu as pltpu
from jax.experimental.pallas import tpu_sc as plsc

N = 10000
E = 160000
D_IN = 128
D_H = 64
D_Z = 16

NW = 32
BATCH = 128
NBATCH = 40
EPAD = NW * NBATCH * BATCH
NPAD = 10112
TROWS = NPAD // 16

MB = 1000
CB = 2048


def _make_deg_kernel():
    mesh = plsc.VectorSubcoreMesh(core_axis_name="c", subcore_axis_name="s")

    @functools.partial(
        pl.kernel,
        out_type=jax.ShapeDtypeStruct((2, NPAD, 16), jnp.float32),
        mesh=mesh,
        scratch_types=[
            pltpu.VMEM((NBATCH, BATCH), jnp.int32),
            pltpu.VMEM((BATCH, 16), jnp.float32),
            pltpu.VMEM_SHARED((NPAD, 16), jnp.float32),
        ],
        compiler_params=pltpu.CompilerParams(use_tc_tiling_on_sc=False),
    )
    def deg_kernel(dst_hbm, ones_hbm, zero_hbm, out_hbm, didx, ones_v, acc):
        c = lax.axis_index("c")
        s = lax.axis_index("s")
        w = c * 16 + s
        r0 = s * TROWS
        pltpu.sync_copy(zero_hbm, acc.at[pl.ds(r0, TROWS)])
        pltpu.sync_copy(dst_hbm.at[pl.ds(w * NBATCH, NBATCH)], didx)
        pltpu.sync_copy(ones_hbm, ones_v)
        plsc.subcore_barrier()

        def body(j, carry):
            pltpu.sync_copy(ones_v, acc.at[didx.at[j]], add=True)
            return carry

        lax.fori_loop(0, NBATCH, body, 0)
        plsc.subcore_barrier()
        pltpu.sync_copy(acc.at[pl.ds(r0, TROWS)],
                        out_hbm.at[c, pl.ds(r0, TROWS)])

    return deg_kernel


def _make_agg_kernel(d):
    mesh = plsc.VectorSubcoreMesh(core_axis_name="c", subcore_axis_name="s")

    @functools.partial(
        pl.kernel,
        out_type=jax.ShapeDtypeStruct((2, NPAD, d), jnp.float32),
        mesh=mesh,
        scratch_types=[
            pltpu.VMEM((NBATCH, BATCH), jnp.int32),
            pltpu.VMEM((NBATCH, BATCH), jnp.int32),
            pltpu.VMEM((BATCH, d), jnp.float32),
            pltpu.VMEM((BATCH, d), jnp.float32),
            pltpu.VMEM_SHARED((NPAD, d), jnp.float32),
            pltpu.SemaphoreType.DMA,
            pltpu.SemaphoreType.DMA,
        ],
        compiler_params=pltpu.CompilerParams(use_tc_tiling_on_sc=False),
    )
    def agg_kernel(hs_hbm, src_hbm, dst_hbm, zero_hbm, out_hbm,
                   sidx, didx, rows0, rows1, acc, sem0, sem1):
        c = lax.axis_index("c")
        s = lax.axis_index("s")
        w = c * 16 + s
        r0 = s * TROWS
        pltpu.sync_copy(zero_hbm, acc.at[pl.ds(r0, TROWS)])
        pltpu.sync_copy(src_hbm.at[pl.ds(w * NBATCH, NBATCH)], sidx)
        pltpu.sync_copy(dst_hbm.at[pl.ds(w * NBATCH, NBATCH)], didx)
        plsc.subcore_barrier()

        rows = (rows0, rows1)
        sems = (sem0, sem1)
        pend = pltpu.async_copy(hs_hbm.at[sidx.at[0]], rows[0], sems[0])
        for j in range(NBATCH):
            b = j % 2
            pend.wait()
            if j + 1 < NBATCH:
                pend = pltpu.async_copy(hs_hbm.at[sidx.at[j + 1]],
                                        rows[1 - b], sems[1 - b])
            pltpu.sync_copy(rows[b], acc.at[didx.at[j]], add=True)
        plsc.subcore_barrier()
        pltpu.sync_copy(acc.at[pl.ds(r0, TROWS)],
                        out_hbm.at[c, pl.ds(r0, TROWS)])

    return agg_kernel


_deg_call = _make_deg_kernel()
_agg64_call = _make_agg_kernel(D_H)
_agg32_call = _make_agg_kernel(2 * D_Z)


def _tc1_body(x_ref, w_ref, b_ref, p0_ref, p1_ref, hs_ref, dinv_ref):
    deg = p0_ref[:, 0:1] + p1_ref[:, 0:1] + 1.0
    dinv = lax.rsqrt(deg)
    h = jnp.dot(x_ref[...], w_ref[...], preferred_element_type=jnp.float32)
    hs_ref[...] = (h + b_ref[...]) * dinv
    dinv_ref[...] = jnp.broadcast_to(dinv, (dinv.shape[0], 16))


def _tc2_body(a0_ref, a1_ref, hs_ref, dinv_ref, w_ref, b_ref, gs_ref):
    dinv = dinv_ref[:, 0:1]
    h1 = jnp.maximum((a0_ref[...] + a1_ref[...] + hs_ref[...]) * dinv, 0.0)
    g = jnp.dot(h1, w_ref[...], preferred_element_type=jnp.float32)
    gs_ref[...] = (g + b_ref[...]) * dinv


def _tc3_body(q0_ref, q1_ref, gs_ref, dinv_ref, eps_ref,
              mu_ref, ls_ref, z_ref):
    m = (q0_ref[...] + q1_ref[...] + gs_ref[...]) * dinv_ref[:, 0:1]
    mu = m[:, :D_Z]
    ls = m[:, D_Z:]
    mu_ref[...] = mu
    ls_ref[...] = ls
    z_ref[...] = mu + eps_ref[...] * jnp.exp(ls)


def _dec_body(zr_ref, zc_ref, out_ref):
    out_ref[...] = lax.dot_general(
        zr_ref[...], zc_ref[...], (((1,), (1,)), ((), ())),
        preferred_element_type=jnp.float32)


def _node_spec(d):
    return pl.BlockSpec((MB, d), lambda i: (i, 0))


def _full_spec(shape):
    return pl.BlockSpec(shape, lambda i: (0,) * len(shape))


def kernel(x, edge_index, W1, b1, Wmu, bmu, Wls, bls, eps):
    src = edge_index[0].astype(jnp.int32)
    dst = edge_index[1].astype(jnp.int32)
    pad = jnp.full((EPAD - E,), N, dtype=jnp.int32)
    srcp = jnp.concatenate([src, pad]).reshape(EPAD // BATCH, BATCH)
    dstp = jnp.concatenate([dst, pad]).reshape(EPAD // BATCH, BATCH)

    ones16 = jnp.ones((BATCH, 16), jnp.float32)
    zero16 = jnp.zeros((TROWS, 16), jnp.float32)
    zero64 = jnp.zeros((TROWS, D_H), jnp.float32)
    zero32 = jnp.zeros((TROWS, 2 * D_Z), jnp.float32)

    degp = _deg_call(dstp, ones16, zero16)
    p0 = degp[0, :N]
    p1 = degp[1, :N]

    grid = N // MB
    hs, dinv = pl.pallas_call(
        _tc1_body,
        grid=(grid,),
        in_specs=[_node_spec(D_IN), _full_spec((D_IN, D_H)),
                  _full_spec((1, D_H)), _node_spec(16), _node_spec(16)],
        out_specs=[_node_spec(D_H), _node_spec(16)],
        out_shape=[jax.ShapeDtypeStruct((N, D_H), jnp.float32),
                   jax.ShapeDtypeStruct((N, 16), jnp.float32)],
    )(x, W1, b1.reshape(1, D_H), p0, p1)

    hs_pad = jnp.pad(hs, ((0, NPAD - N), (0, 0)))
    agg1 = _agg64_call(hs_pad, srcp, dstp, zero64)
    a0 = agg1[0, :N]
    a1 = agg1[1, :N]

    Wc = jnp.concatenate([Wmu, Wls], axis=1)
    bc = jnp.concatenate([bmu, bls]).reshape(1, 2 * D_Z)
    gs = pl.pallas_call(
        _tc2_body,
        grid=(grid,),
        in_specs=[_node_spec(D_H), _node_spec(D_H), _node_spec(D_H),
                  _node_spec(16), _full_spec((D_H, 2 * D_Z)),
                  _full_spec((1, 2 * D_Z))],
        out_specs=_node_spec(2 * D_Z),
        out_shape=jax.ShapeDtypeStruct((N, 2 * D_Z), jnp.float32),
    )(a0, a1, hs, dinv, Wc, bc)

    gs_pad = jnp.pad(gs, ((0, NPAD - N), (0, 0)))
    agg2 = _agg32_call(gs_pad, srcp, dstp, zero32)
    q0 = agg2[0, :N]
    q1 = agg2[1, :N]

    mu, logstd, z = pl.pallas_call(
        _tc3_body,
        grid=(grid,),
        in_specs=[_node_spec(2 * D_Z), _node_spec(2 * D_Z),
                  _node_spec(2 * D_Z), _node_spec(16), _node_spec(D_Z)],
        out_specs=[_node_spec(D_Z), _node_spec(D_Z), _node_spec(D_Z)],
        out_shape=[jax.ShapeDtypeStruct((N, D_Z), jnp.float32),
                   jax.ShapeDtypeStruct((N, D_Z), jnp.float32),
                   jax.ShapeDtypeStruct((N, D_Z), jnp.float32)],
    )(q0, q1, gs, dinv, eps)

    adj = pl.pallas_call(
        _dec_body,
        grid=(N // MB, pl.cdiv(N, CB)),
        in_specs=[pl.BlockSpec((MB, D_Z), lambda i, j: (i, 0)),
                  pl.BlockSpec((CB, D_Z), lambda i, j: (j, 0))],
        out_specs=pl.BlockSpec((MB, CB), lambda i, j: (i, j)),
        out_shape=jax.ShapeDtypeStruct((N, N), jnp.float32),
    )(z, z)

    return (adj, mu, logstd)


# speedup vs baseline: 13.0475x; 1.0513x over previous
import functools

import jax
import jax.numpy as jnp
from jax import lax
from jax.experimental import pallas as pl
from jax.experimental.pallas import tpu as pltpu
from jax.experimental.pallas import tpu_sc as plsc

N = 10000
E = 160000
D_IN = 128
D_H = 64
D_Z = 16

NW = 32
NBUF = 4
BATCH = 128
NBATCH = 40
EPAD = NW * NBATCH * BATCH
NPAD = 10112
TROWS = NPAD // 16

MB = 1000
CB = 2048


def _make_deg_kernel():
    mesh = plsc.VectorSubcoreMesh(core_axis_name="c", subcore_axis_name="s")

    @functools.partial(
        pl.kernel,
        out_type=jax.ShapeDtypeStruct((2, NPAD, 16), jnp.float32),
        mesh=mesh,
        scratch_types=[
            pltpu.VMEM((NBATCH, BATCH), jnp.int32),
            pltpu.VMEM((BATCH, 16), jnp.float32),
            pltpu.VMEM_SHARED((NPAD, 16), jnp.float32),
        ],
        compiler_params=pltpu.CompilerParams(use_tc_tiling_on_sc=False),
    )
    def deg_kernel(dst_hbm, ones_hbm, zero_hbm, out_hbm, didx, ones_v, acc):
        c = lax.axis_index("c")
        s = lax.axis_index("s")
        w = c * 16 + s
        r0 = s * TROWS
        pltpu.sync_copy(zero_hbm, acc.at[pl.ds(r0, TROWS)])
        pltpu.sync_copy(dst_hbm.at[pl.ds(w * NBATCH, NBATCH)], didx)
        pltpu.sync_copy(ones_hbm, ones_v)
        plsc.subcore_barrier()

        def body(j, carry):
            pltpu.sync_copy(ones_v, acc.at[didx.at[j]], add=True)
            return carry

        lax.fori_loop(0, NBATCH, body, 0)
        plsc.subcore_barrier()
        pltpu.sync_copy(acc.at[pl.ds(r0, TROWS)],
                        out_hbm.at[c, pl.ds(r0, TROWS)])

    return deg_kernel


def _make_agg_kernel(d):
    mesh = plsc.VectorSubcoreMesh(core_axis_name="c", subcore_axis_name="s")

    @functools.partial(
        pl.kernel,
        out_type=jax.ShapeDtypeStruct((2, NPAD, d), jnp.float32),
        mesh=mesh,
        scratch_types=[
            pltpu.VMEM((NBATCH, BATCH), jnp.int32),
            pltpu.VMEM((NBATCH, BATCH), jnp.int32),
            [pltpu.VMEM((BATCH, d), jnp.float32)] * NBUF,
            pltpu.VMEM_SHARED((NPAD, d), jnp.float32),
            [pltpu.SemaphoreType.DMA] * NBUF,
        ],
        compiler_params=pltpu.CompilerParams(use_tc_tiling_on_sc=False),
    )
    def agg_kernel(hs_hbm, src_hbm, dst_hbm, zero_hbm, out_hbm,
                   sidx, didx, rows, acc, sems):
        c = lax.axis_index("c")
        s = lax.axis_index("s")
        w = c * 16 + s
        r0 = s * TROWS
        pltpu.sync_copy(zero_hbm, acc.at[pl.ds(r0, TROWS)])
        pltpu.sync_copy(src_hbm.at[pl.ds(w * NBATCH, NBATCH)], sidx)
        pltpu.sync_copy(dst_hbm.at[pl.ds(w * NBATCH, NBATCH)], didx)
        plsc.subcore_barrier()

        pend = [None] * NBUF
        for j in range(NBUF - 1):
            pend[j % NBUF] = pltpu.async_copy(
                hs_hbm.at[sidx.at[j]], rows[j % NBUF], sems[j % NBUF])
        for j in range(NBATCH):
            b = j % NBUF
            pend[b].wait()
            nxt = j + NBUF - 1
            if nxt < NBATCH:
                nb = nxt % NBUF
                pend[nb] = pltpu.async_copy(hs_hbm.at[sidx.at[nxt]],
                                            rows[nb], sems[nb])
            pltpu.sync_copy(rows[b], acc.at[didx.at[j]], add=True)
        plsc.subcore_barrier()
        pltpu.sync_copy(acc.at[pl.ds(r0, TROWS)],
                        out_hbm.at[c, pl.ds(r0, TROWS)])

    return agg_kernel


_deg_call = _make_deg_kernel()
_agg64_call = _make_agg_kernel(D_H)
_agg32_call = _make_agg_kernel(2 * D_Z)


def _tc1_body(x_ref, w_ref, b_ref, p0_ref, p1_ref, hs_ref, dinv_ref):
    deg = p0_ref[:, 0:1] + p1_ref[:, 0:1] + 1.0
    dinv = lax.rsqrt(deg)
    h = jnp.dot(x_ref[...], w_ref[...], preferred_element_type=jnp.float32)
    hs_ref[...] = (h + b_ref[...]) * dinv
    dinv_ref[...] = jnp.broadcast_to(dinv, (dinv.shape[0], 16))


def _tc2_body(a0_ref, a1_ref, hs_ref, dinv_ref, w_ref, b_ref, gs_ref):
    dinv = dinv_ref[:, 0:1]
    h1 = jnp.maximum((a0_ref[...] + a1_ref[...] + hs_ref[...]) * dinv, 0.0)
    g = jnp.dot(h1, w_ref[...], preferred_element_type=jnp.float32)
    gs_ref[...] = (g + b_ref[...]) * dinv


def _tc3_body(q0_ref, q1_ref, gs_ref, dinv_ref, eps_ref,
              mu_ref, ls_ref, z_ref):
    m = (q0_ref[...] + q1_ref[...] + gs_ref[...]) * dinv_ref[:, 0:1]
    mu = m[:, :D_Z]
    ls = m[:, D_Z:]
    mu_ref[...] = mu
    ls_ref[...] = ls
    z_ref[...] = mu + eps_ref[...] * jnp.exp(ls)


def _dec_body(zr_ref, zc_ref, out_ref):
    out_ref[...] = lax.dot_general(
        zr_ref[...], zc_ref[...], (((1,), (1,)), ((), ())),
        preferred_element_type=jnp.float32)


def _node_spec(d):
    return pl.BlockSpec((MB, d), lambda i: (i, 0))


def _full_spec(shape):
    return pl.BlockSpec(shape, lambda i: (0,) * len(shape))


def kernel(x, edge_index, W1, b1, Wmu, bmu, Wls, bls, eps):
    src = edge_index[0].astype(jnp.int32)
    dst = edge_index[1].astype(jnp.int32)
    pad = jnp.full((EPAD - E,), N, dtype=jnp.int32)
    srcp = jnp.concatenate([src, pad]).reshape(EPAD // BATCH, BATCH)
    dstp = jnp.concatenate([dst, pad]).reshape(EPAD // BATCH, BATCH)

    ones16 = jnp.ones((BATCH, 16), jnp.float32)
    zero16 = jnp.zeros((TROWS, 16), jnp.float32)
    zero64 = jnp.zeros((TROWS, D_H), jnp.float32)
    zero32 = jnp.zeros((TROWS, 2 * D_Z), jnp.float32)

    degp = _deg_call(dstp, ones16, zero16)
    p0 = degp[0, :N]
    p1 = degp[1, :N]

    grid = N // MB
    hs, dinv = pl.pallas_call(
        _tc1_body,
        grid=(grid,),
        in_specs=[_node_spec(D_IN), _full_spec((D_IN, D_H)),
                  _full_spec((1, D_H)), _node_spec(16), _node_spec(16)],
        out_specs=[_node_spec(D_H), _node_spec(16)],
        out_shape=[jax.ShapeDtypeStruct((N, D_H), jnp.float32),
                   jax.ShapeDtypeStruct((N, 16), jnp.float32)],
    )(x, W1, b1.reshape(1, D_H), p0, p1)

    hs_pad = jnp.pad(hs, ((0, NPAD - N), (0, 0)))
    agg1 = _agg64_call(hs_pad, srcp, dstp, zero64)
    a0 = agg1[0, :N]
    a1 = agg1[1, :N]

    Wc = jnp.concatenate([Wmu, Wls], axis=1)
    bc = jnp.concatenate([bmu, bls]).reshape(1, 2 * D_Z)
    gs = pl.pallas_call(
        _tc2_body,
        grid=(grid,),
        in_specs=[_node_spec(D_H), _node_spec(D_H), _node_spec(D_H),
                  _node_spec(16), _full_spec((D_H, 2 * D_Z)),
                  _full_spec((1, 2 * D_Z))],
        out_specs=_node_spec(2 * D_Z),
        out_shape=jax.ShapeDtypeStruct((N, 2 * D_Z), jnp.float32),
    )(a0, a1, hs, dinv, Wc, bc)

    gs_pad = jnp.pad(gs, ((0, NPAD - N), (0, 0)))
    agg2 = _agg32_call(gs_pad, srcp, dstp, zero32)
    q0 = agg2[0, :N]
    q1 = agg2[1, :N]

    mu, logstd, z = pl.pallas_call(
        _tc3_body,
        grid=(grid,),
        in_specs=[_node_spec(2 * D_Z), _node_spec(2 * D_Z),
                  _node_spec(2 * D_Z), _node_spec(16), _node_spec(D_Z)],
        out_specs=[_node_spec(D_Z), _node_spec(D_Z), _node_spec(D_Z)],
        out_shape=[jax.ShapeDtypeStruct((N, D_Z), jnp.float32),
                   jax.ShapeDtypeStruct((N, D_Z), jnp.float32),
                   jax.ShapeDtypeStruct((N, D_Z), jnp.float32)],
    )(q0, q1, gs, dinv, eps)

    adj = pl.pallas_call(
        _dec_body,
        grid=(N // MB, pl.cdiv(N, CB)),
        in_specs=[pl.BlockSpec((MB, D_Z), lambda i, j: (i, 0)),
                  pl.BlockSpec((CB, D_Z), lambda i, j: (j, 0))],
        out_specs=pl.BlockSpec((MB, CB), lambda i, j: (i, j)),
        out_shape=jax.ShapeDtypeStruct((N, N), jnp.float32),
    )(z, z)

    return (adj, mu, logstd)
